# baseline (speedup 1.0000x reference)
import functools

import jax
import jax.numpy as jnp
from jax import lax
from jax.experimental import pallas as pl
from jax.experimental.pallas import tpu as pltpu
from jax.experimental.pallas import tpu_sc as plsc

_B = 16384
_F = 26
_K = 16
_NIDX = _B * _F
_NC, _NS = 2, 16
_NW = _NC * _NS
_PER_W = _NIDX // _NW
_CH = 1664
_NSTEP = _PER_W // _CH

_R = 1024


def _sc_gather(xf, emb, fc):
    mesh = plsc.VectorSubcoreMesh(core_axis_name="c", subcore_axis_name="s")

    @functools.partial(
        pl.kernel,
        mesh=mesh,
        compiler_params=pltpu.CompilerParams(use_tc_tiling_on_sc=False),
        out_type=(
            jax.ShapeDtypeStruct((_NIDX, _K), jnp.float32),
            jax.ShapeDtypeStruct((_NIDX, 1), jnp.float32),
        ),
        scratch_types=[
            pltpu.VMEM((_CH,), jnp.int32),
            pltpu.VMEM((_CH, _K), jnp.float32),
            pltpu.VMEM((_CH, 1), jnp.float32),
        ],
    )
    def k(x_hbm, emb_hbm, fc_hbm, e_out, f_out, idxb, ebuf, fbuf):
        wid = lax.axis_index("s") * _NC + lax.axis_index("c")
        base = wid * _PER_W
        for step in range(_NSTEP):
            off = base + step * _CH
            pltpu.sync_copy(x_hbm.at[pl.ds(off, _CH)], idxb)
            pltpu.sync_copy(emb_hbm.at[idxb], ebuf)
            pltpu.sync_copy(ebuf, e_out.at[pl.ds(off, _CH)])
            pltpu.sync_copy(fc_hbm.at[idxb], fbuf)
            pltpu.sync_copy(fbuf, f_out.at[pl.ds(off, _CH)])

    return k(xf, emb, fc)


def _tc_body(e_ref, fc_ref, w_ref, b_ref, o_ref):
    eb = e_ref[...]
    fcb = fc_ref[...]
    m = (lax.broadcasted_iota(jnp.int32, (_F * _K, _K), 0) % _K
         == lax.broadcasted_iota(jnp.int32, (_F * _K, _K), 1)).astype(jnp.float32)
    s = jax.lax.dot(eb, m, precision=jax.lax.Precision.HIGHEST,
                    preferred_element_type=jnp.float32)
    sos = jnp.sum(s * s, axis=1)
    ssq = jnp.sum(eb * eb, axis=1)
    fcs = jnp.sum(fcb, axis=1)
    w = w_ref[0, 0]
    bb = b_ref[0]
    z = fcs * w + bb + 0.5 * (sos - ssq)
    o_ref[...] = jax.nn.sigmoid(z)[:, None]


def _tc_reduce(e2, fc2, W, b):
    return pl.pallas_call(
        _tc_body,
        grid=(_B // _R,),
        in_specs=[
            pl.BlockSpec((_R, _F * _K), lambda i: (i, 0)),
            pl.BlockSpec((_R, _F), lambda i: (i, 0)),
            pl.BlockSpec(memory_space=pltpu.SMEM),
            pl.BlockSpec(memory_space=pltpu.SMEM),
        ],
        out_specs=pl.BlockSpec((_R, 1), lambda i: (i, 0)),
        out_shape=jax.ShapeDtypeStruct((_B, 1), jnp.float32),
    )(e2, fc2, W, b)


def kernel(x, emb_table, fc_table, W, b):
    xf = x.reshape(_NIDX)
    e_flat, f_flat = _sc_gather(xf, emb_table, fc_table)
    out = _tc_reduce(e_flat.reshape(_B, _F * _K), f_flat.reshape(_B, _F), W, b)
    return out.reshape(_B)


# speedup vs baseline: 2.8342x; 2.8342x over previous
import functools

import jax
import jax.numpy as jnp
from jax import lax
from jax.experimental import pallas as pl
from jax.experimental.pallas import tpu as pltpu
from jax.experimental.pallas import tpu_sc as plsc

_N = 1000012
_B = 16384
_F = 26
_K = 16
_NIDX = _B * _F
_NC, _NS = 2, 16
_NW = _NC * _NS
_PER_W = _NIDX // _NW
_CH = 1664
_RCH = _CH // _F
_NSTEP = _PER_W // _CH

_TC_C = 16384
_R = 2048


def _sc_fm(xf, emb_rm, fc1):
    mesh = plsc.VectorSubcoreMesh(core_axis_name="c", subcore_axis_name="s")

    @functools.partial(
        pl.kernel,
        mesh=mesh,
        compiler_params=pltpu.CompilerParams(use_tc_tiling_on_sc=False),
        out_type=(
            jax.ShapeDtypeStruct((_B, _K), jnp.float32),
            jax.ShapeDtypeStruct((_NIDX,), jnp.float32),
        ),
        scratch_types=[
            pltpu.VMEM((_CH,), jnp.int32),
            pltpu.VMEM((_CH, _K), jnp.float32),
            pltpu.VMEM((_CH,), jnp.float32),
            pltpu.VMEM((_RCH, _K), jnp.float32),
        ],
    )
    def k(x_hbm, emb_hbm, fc_hbm, t_out, f_out, idxb, ebuf, fbuf, tbuf):
        wid = lax.axis_index("s") * _NC + lax.axis_index("c")
        base = wid * _PER_W
        rbase = wid * (_PER_W // _F)
        for step in range(_NSTEP):
            j0 = base + step * _CH
            r0 = rbase + step * _RCH
            pltpu.sync_copy(x_hbm.at[pl.ds(j0, _CH)], idxb)
            pltpu.sync_copy(emb_hbm.at[idxb], ebuf)
            pltpu.sync_copy(fc_hbm.at[idxb], fbuf)
            pltpu.sync_copy(fbuf, f_out.at[pl.ds(j0, _CH)])

            @pl.loop(0, _RCH)
            def _(r):
                p = r * _F
                s = ebuf[p, :]
                ss = s * s
                for f in range(1, _F):
                    v = ebuf[p + f, :]
                    s = s + v
                    ss = ss + v * v
                tbuf[r, :] = s * s - ss

            pltpu.sync_copy(tbuf, t_out.at[pl.ds(r0, _RCH)])

    return k(xf, emb_rm, fc1)


def _fin_body(t_ref, fc_ref, w_ref, b_ref, o_ref):
    inter = 0.5 * jnp.sum(t_ref[...], axis=1)
    fcs = jnp.sum(fc_ref[...], axis=1)
    z = fcs * w_ref[0, 0] + b_ref[0] + inter
    o_ref[...] = jax.nn.sigmoid(z)


def _tc_finish(t2, fc2, W, b):
    return pl.pallas_call(
        _fin_body,
        grid=(_B // _R,),
        in_specs=[
            pl.BlockSpec((_R, _K), lambda i: (i, 0)),
            pl.BlockSpec((_R, _F), lambda i: (i, 0)),
            pl.BlockSpec(memory_space=pltpu.SMEM),
            pl.BlockSpec(memory_space=pltpu.SMEM),
        ],
        out_specs=pl.BlockSpec((_R,), lambda i: (i,)),
        out_shape=jax.ShapeDtypeStruct((_B,), jnp.float32),
        compiler_params=pltpu.CompilerParams(
            dimension_semantics=("parallel",)),
    )(t2, fc2, W, b)


def kernel(x, emb_table, fc_table, W, b):
    emb_rm = emb_table.reshape(_N * _K).reshape(_N, _K)
    fc1 = fc_table.reshape(_N)
    xf = x.reshape(_NIDX)
    t2, fcv = _sc_fm(xf, emb_rm, fc1)
    return _tc_finish(t2, fcv.reshape(_B, _F), W, b)
